# Initial kernel scaffold; baseline (speedup 1.0000x reference)
import jax
import jax.numpy as jnp
from jax.experimental import pallas as pl


def kernel(x, edge_index, W, b):
    raise NotImplementedError("write your pallas kernel here")


# speedup vs baseline: 10.9061x; 10.9061x over previous
import functools

import jax
import jax.numpy as jnp
from jax import lax
from jax.experimental import pallas as pl
from jax.experimental.pallas import tpu as pltpu
from jax.experimental.pallas import tpu_sc as plsc

N = 10000
E = 320000
F = 128

N_PAD = 10112
RPT = N_PAD // 16

TILES = 32
CW = 128
CH = 80
EPAD = TILES * CH * CW

_mesh = plsc.VectorSubcoreMesh(core_axis_name="c", subcore_axis_name="s")
_sc_params = pltpu.CompilerParams(needs_layout_passes=False)


@functools.partial(
    pl.kernel,
    mesh=_mesh,
    compiler_params=_sc_params,
    out_type=jax.ShapeDtypeStruct((TILES, N_PAD), jnp.float32),
    scratch_types=[
        pltpu.VMEM((CH, CW), jnp.int32),
        pltpu.VMEM((N_PAD,), jnp.float32),
    ],
)
def _deg_kernel(dst_hbm, out_hbm, idx_v, deg_v):
    cid = lax.axis_index("c")
    sid = lax.axis_index("s")
    t = cid * 16 + sid

    pltpu.sync_copy(dst_hbm.at[t], idx_v)

    def _zero(r, _):
        deg_v[pl.ds(r * 16, 16)] = jnp.zeros((16,), jnp.float32)
        return _
    lax.fori_loop(0, N_PAD // 16, _zero, None)

    def _scat(j, _):
        ones = jnp.full((16,), 1.0, jnp.float32)
        for k in range(CW // 16):
            v = idx_v[j, pl.ds(k * 16, 16)]
            plsc.addupdate_scatter(deg_v, [v], ones)
        return _
    lax.fori_loop(0, CH, _scat, None)

    pltpu.sync_copy(deg_v, out_hbm.at[t])


def _lin_body(dp_ref, x_ref, w_ref, y_ref, dinv_ref):
    deg = jnp.sum(dp_ref[...], axis=1, keepdims=True) + 1.0
    dinv = lax.rsqrt(deg)
    xw = jnp.dot(x_ref[...], w_ref[...], preferred_element_type=jnp.float32)
    y_ref[...] = xw * dinv
    dinv_ref[...] = dinv


_lin_call = pl.pallas_call(
    _lin_body,
    out_shape=(
        jax.ShapeDtypeStruct((N_PAD, F), jnp.float32),
        jax.ShapeDtypeStruct((N_PAD, 1), jnp.float32),
    ),
)


@functools.partial(
    pl.kernel,
    mesh=_mesh,
    compiler_params=_sc_params,
    out_type=jax.ShapeDtypeStruct((2, N_PAD, F), jnp.float32),
    scratch_types=[
        pltpu.VMEM((CH, CW), jnp.int32),
        pltpu.VMEM((CH, CW), jnp.int32),
        pltpu.VMEM((CW, F), jnp.float32),
        pltpu.VMEM_SHARED((N_PAD, F), jnp.float32),
        pltpu.SemaphoreType.DMA,
    ],
)
def _agg_kernel(y_hbm, src_hbm, dst_hbm, out_hbm, idx_s, idx_d, rows, acc, sem):
    cid = lax.axis_index("c")
    sid = lax.axis_index("s")
    t = cid * 16 + sid

    pltpu.sync_copy(src_hbm.at[t], idx_s)
    pltpu.sync_copy(dst_hbm.at[t], idx_d)

    def _zero(r, _):
        for k in range(F // 16):
            rows[r, pl.ds(k * 16, 16)] = jnp.zeros((16,), jnp.float32)
        return _
    lax.fori_loop(0, CW, _zero, None)

    base = sid * RPT
    for q in range(4):
        pltpu.sync_copy(rows, acc.at[pl.ds(base + q * CW, CW)])
    pltpu.sync_copy(rows.at[pl.ds(0, RPT - 4 * CW)],
                    acc.at[pl.ds(base + 4 * CW, RPT - 4 * CW)])
    plsc.subcore_barrier()

    def _edge(j, _):
        pltpu.async_copy(y_hbm.at[idx_s.at[j]], rows, sem).wait()
        pltpu.sync_copy(rows, acc.at[idx_d.at[j]], add=True)
        return _
    lax.fori_loop(0, CH, _edge, None)

    plsc.subcore_barrier()
    for q in range(4):
        pltpu.sync_copy(acc.at[pl.ds(base + q * CW, CW)], rows)
        pltpu.sync_copy(rows, out_hbm.at[cid, pl.ds(base + q * CW, CW)])
    pltpu.sync_copy(acc.at[pl.ds(base + 4 * CW, RPT - 4 * CW)],
                    rows.at[pl.ds(0, RPT - 4 * CW)])
    pltpu.sync_copy(rows.at[pl.ds(0, RPT - 4 * CW)],
                    out_hbm.at[cid, pl.ds(base + 4 * CW, RPT - 4 * CW)])


def _fin_body(a_ref, y_ref, dinv_ref, b_ref, o_ref):
    s = (a_ref[0] + a_ref[1] + y_ref[...]) * dinv_ref[...] + b_ref[...]
    o_ref[...] = jnp.maximum(s, 0.0)


_fin_call = pl.pallas_call(
    _fin_body,
    out_shape=jax.ShapeDtypeStruct((N_PAD, F), jnp.float32),
)


def kernel(x, edge_index, W, b):
    src = edge_index[0].astype(jnp.int32)
    dst = edge_index[1].astype(jnp.int32)
    pad = EPAD - E
    src_p = jnp.concatenate([src, jnp.full((pad,), N, jnp.int32)]).reshape(TILES, CH, CW)
    dst_p = jnp.concatenate([dst, jnp.full((pad,), N, jnp.int32)]).reshape(TILES, CH, CW)
    x_p = jnp.pad(x, ((0, N_PAD - N), (0, 0)))

    dp = _deg_kernel(dst_p)
    y, dinv = _lin_call(dp.T, x_p, W)
    agg = _agg_kernel(y, src_p, dst_p)
    out = _fin_call(agg, y, dinv, b.reshape(1, F))
    return out[:N]


# speedup vs baseline: 12.1383x; 1.1130x over previous
import functools

import jax
import jax.numpy as jnp
from jax import lax
from jax.experimental import pallas as pl
from jax.experimental.pallas import tpu as pltpu
from jax.experimental.pallas import tpu_sc as plsc

N = 10000
E = 320000
F = 128

N_PAD = 10112
RPT = N_PAD // 16

TILES = 32
CW = 64
CH = 160
EPAD = TILES * CH * CW

_mesh = plsc.VectorSubcoreMesh(core_axis_name="c", subcore_axis_name="s")
_sc_params = pltpu.CompilerParams(needs_layout_passes=False,
                                  use_tc_tiling_on_sc=False)


@functools.partial(
    pl.kernel,
    mesh=_mesh,
    compiler_params=_sc_params,
    out_type=jax.ShapeDtypeStruct((TILES, N_PAD), jnp.float32),
    scratch_types=[
        pltpu.VMEM((CH, CW), jnp.int32),
        pltpu.VMEM((N_PAD,), jnp.float32),
    ],
)
def _deg_kernel(dst_hbm, out_hbm, idx_v, deg_v):
    cid = lax.axis_index("c")
    sid = lax.axis_index("s")
    t = cid * 16 + sid

    pltpu.sync_copy(dst_hbm.at[t], idx_v)

    def _zero(r, _):
        deg_v[pl.ds(r * 16, 16)] = jnp.zeros((16,), jnp.float32)
        return _
    lax.fori_loop(0, N_PAD // 16, _zero, None)

    def _scat(j, _):
        ones = jnp.full((16,), 1.0, jnp.float32)
        for k in range(CW // 16):
            v = idx_v[j, pl.ds(k * 16, 16)]
            plsc.addupdate_scatter(deg_v, [v], ones)
        return _
    lax.fori_loop(0, CH, _scat, None)

    pltpu.sync_copy(deg_v, out_hbm.at[t])


def _lin_body(dp_ref, x_ref, w_ref, y_ref, dinv_ref):
    deg = jnp.sum(dp_ref[...], axis=1, keepdims=True) + 1.0
    dinv = lax.rsqrt(deg)
    xw = jnp.dot(x_ref[...], w_ref[...], preferred_element_type=jnp.float32)
    y_ref[...] = xw * dinv
    dinv_ref[...] = dinv


_lin_call = pl.pallas_call(
    _lin_body,
    out_shape=(
        jax.ShapeDtypeStruct((N_PAD, F), jnp.float32),
        jax.ShapeDtypeStruct((N_PAD, 1), jnp.float32),
    ),
)


@functools.partial(
    pl.kernel,
    mesh=_mesh,
    compiler_params=_sc_params,
    out_type=jax.ShapeDtypeStruct((2, N_PAD, F), jnp.float32),
    scratch_types=[
        pltpu.VMEM((CH, CW), jnp.int32),
        pltpu.VMEM((CH, CW), jnp.int32),
        pltpu.VMEM((2, CW, F), jnp.float32),
        pltpu.VMEM_SHARED((N_PAD, F), jnp.float32),
        pltpu.SemaphoreType.DMA,
        pltpu.SemaphoreType.DMA,
    ],
)
def _agg_kernel(y_hbm, src_hbm, dst_hbm, out_hbm, idx_s, idx_d, rows, acc,
                sem0, sem1):
    cid = lax.axis_index("c")
    sid = lax.axis_index("s")
    t = cid * 16 + sid

    pltpu.sync_copy(src_hbm.at[t], idx_s)
    pltpu.sync_copy(dst_hbm.at[t], idx_d)

    def _zero(r, _):
        for k in range(F // 16):
            rows[0, r, pl.ds(k * 16, 16)] = jnp.zeros((16,), jnp.float32)
        return _
    lax.fori_loop(0, CW, _zero, None)

    base = sid * RPT
    nfull, rem = divmod(RPT, CW)
    for q in range(nfull):
        pltpu.sync_copy(rows.at[0], acc.at[pl.ds(base + q * CW, CW)])
    pltpu.sync_copy(rows.at[0, pl.ds(0, rem)],
                    acc.at[pl.ds(base + nfull * CW, rem)])
    plsc.subcore_barrier()

    pltpu.async_copy(y_hbm.at[idx_s.at[0]], rows.at[0], sem0)

    def _pair(k, _):
        j0 = 2 * k
        pltpu.async_copy(y_hbm.at[idx_s.at[j0 + 1]], rows.at[1], sem1)
        pltpu.make_async_copy(y_hbm.at[idx_s.at[j0]], rows.at[0], sem0).wait()
        pltpu.sync_copy(rows.at[0], acc.at[idx_d.at[j0]], add=True)

        @pl.when(j0 + 2 < CH)
        def _():
            pltpu.async_copy(y_hbm.at[idx_s.at[j0 + 2]], rows.at[0], sem0)
        pltpu.make_async_copy(y_hbm.at[idx_s.at[j0 + 1]], rows.at[1], sem1).wait()
        pltpu.sync_copy(rows.at[1], acc.at[idx_d.at[j0 + 1]], add=True)
        return _
    lax.fori_loop(0, CH // 2, _pair, None)

    plsc.subcore_barrier()
    for q in range(nfull):
        pltpu.sync_copy(acc.at[pl.ds(base + q * CW, CW)], rows.at[0])
        pltpu.sync_copy(rows.at[0], out_hbm.at[cid, pl.ds(base + q * CW, CW)])
    pltpu.sync_copy(acc.at[pl.ds(base + nfull * CW, rem)],
                    rows.at[0, pl.ds(0, rem)])
    pltpu.sync_copy(rows.at[0, pl.ds(0, rem)],
                    out_hbm.at[cid, pl.ds(base + nfull * CW, rem)])


def _fin_body(a_ref, y_ref, dinv_ref, b_ref, o_ref):
    s = (a_ref[0] + a_ref[1] + y_ref[...]) * dinv_ref[...] + b_ref[...]
    o_ref[...] = jnp.maximum(s, 0.0)


_fin_call = pl.pallas_call(
    _fin_body,
    out_shape=jax.ShapeDtypeStruct((N_PAD, F), jnp.float32),
)


def kernel(x, edge_index, W, b):
    src = edge_index[0].astype(jnp.int32)
    dst = edge_index[1].astype(jnp.int32)
    pad = EPAD - E
    src_p = jnp.concatenate([src, jnp.full((pad,), N, jnp.int32)]).reshape(TILES, CH, CW)
    dst_p = jnp.concatenate([dst, jnp.full((pad,), N, jnp.int32)]).reshape(TILES, CH, CW)
    x_p = jnp.pad(x, ((0, N_PAD - N), (0, 0)))

    dp = _deg_kernel(dst_p)
    y, dinv = _lin_call(dp.T, x_p, W)
    agg = _agg_kernel(y, src_p, dst_p)
    out = _fin_call(agg, y, dinv, b.reshape(1, F))
    return out[:N]
